# baseline (speedup 1.0000x reference)
import functools
import math

import jax
import jax.numpy as jnp
from jax import lax
from jax.experimental import pallas as pl
from jax.experimental.pallas import tpu as pltpu
from jax.experimental.pallas import tpu_sc as plsc

NC = 2
NS = 16
L = 16


def _largest_div(n, cap, mult=8):
    for b in range(min(n, cap), 0, -1):
        if n % b == 0 and b % mult == 0:
            return b
    raise ValueError((n, cap, mult))


def _proj_body(x_ref, w_ref, h_ref):
    h_ref[...] = jnp.dot(x_ref[...], w_ref[0],
                         preferred_element_type=jnp.float32)


def _projections(x, w_low, w_high):
    n, d = x.shape
    b = _largest_div(n, 1024)
    nb = n // b
    w_stack = jnp.stack([w_low, w_high])
    return pl.pallas_call(
        _proj_body,
        grid=(2, nb),
        in_specs=[pl.BlockSpec((b, d), lambda p, i: (i, 0)),
                  pl.BlockSpec((1, d, d), lambda p, i: (p, 0, 0))],
        out_specs=pl.BlockSpec((b, d), lambda p, i: (p * nb + i, 0)),
        out_shape=jax.ShapeDtypeStruct((2 * n, d), jnp.float32),
    )(x, w_stack)


def _lane_bcast(vec, t):
    return lax.gather(
        vec, jnp.full((L, 1), t, jnp.int32),
        lax.GatherDimensionNumbers(offset_dims=(), collapsed_slice_dims=(0,),
                                   start_index_map=(0,)),
        slice_sizes=(1,),
        mode=lax.GatherScatterMode.PROMISE_IN_BOUNDS)


def _spmm_sc(h_stack, rowcol, vals_stack):
    n2, d = h_stack.shape
    n = n2 // 2
    e = rowcol.shape[0] // 2
    ep = e // NS
    k = _largest_div(ep, 128)
    nch = ep // k
    npad = -(-n // (NS * 8)) * (NS * 8)
    rps = npad // NS
    zr = _largest_div(rps, 64)
    nj = d // L

    mesh = plsc.VectorSubcoreMesh(core_axis_name="c", subcore_axis_name="s",
                                  num_cores=NC, num_subcores=NS)

    @functools.partial(
        pl.kernel,
        out_type=jax.ShapeDtypeStruct((2 * npad, d), jnp.float32),
        mesh=mesh,
        scratch_types=[
            pltpu.VMEM((4, k), jnp.int32),
            pltpu.VMEM((4, k), jnp.int32),
            pltpu.VMEM((4, k), jnp.float32),
            pltpu.VMEM((k, d), jnp.float32),
            pltpu.VMEM((k, d), jnp.float32),
            pltpu.VMEM((zr, d), jnp.float32),
            pltpu.VMEM_SHARED((npad, d), jnp.float32),
            pltpu.SemaphoreType.DMA,
            pltpu.SemaphoreType.DMA,
            pltpu.SemaphoreType.DMA,
            pltpu.SemaphoreType.DMA,
            pltpu.SemaphoreType.DMA,
            pltpu.SemaphoreType.DMA,
        ],
    )
    def spmm_kernel(h_ref, rowcol_ref, vals_ref, out_ref,
                    idx4, rdx4, val4, rows0, rows1,
                    zero_v, acc, esem0, esem1, gsem0, gsem1, ssem0, ssem1):
        c = lax.axis_index("c")
        s = lax.axis_index("s")
        rows = (rows0, rows1)
        esem = (esem0, esem1)
        gsem = (gsem0, gsem1)
        ssem = (ssem0, ssem1)

        def start_edges(g, b):
            base = s * ep + g * k
            q = lax.rem(g, 4)
            pltpu.async_copy(rowcol_ref.at[pl.ds(e + base, k)], idx4.at[q],
                             esem[b])
            pltpu.async_copy(rowcol_ref.at[pl.ds(base, k)], rdx4.at[q],
                             esem[b])
            pltpu.async_copy(vals_ref.at[pl.ds(c * e + base, k)], val4.at[q],
                             esem[b])

        def wait_edges(b):
            for dst in (idx4, rdx4, val4):
                pltpu.make_async_copy(rowcol_ref.at[pl.ds(0, k)], dst.at[0],
                                      esem[b]).wait()

        def shift_idx(g):
            q = lax.rem(g, 4)
            off = c * n

            def sbody(gg, carry):
                sl = pl.ds(gg * L, L)
                idx4[q, sl] = idx4[q, sl] + off
                return carry

            lax.fori_loop(0, k // L, sbody, 0)

        def start_gather(g, b):
            pltpu.async_copy(h_ref.at[idx4.at[lax.rem(g, 4)]], rows[b],
                             gsem[b])

        def wait_gather(b):
            pltpu.make_async_copy(h_ref.at[idx4.at[0]], rows[b],
                                  gsem[b]).wait()

        def start_scatter(g, b):
            pltpu.async_copy(rows[b], acc.at[rdx4.at[lax.rem(g, 4)]],
                             ssem[b], add=True)

        def wait_scatter(b):
            pltpu.make_async_copy(rows[b], acc.at[rdx4.at[0]], ssem[b]).wait()

        def scale(g, b):
            q = lax.rem(g, 4)

            def grp(gg, carry):
                val16 = val4[q, pl.ds(gg * L, L)]
                for t in range(L):
                    vv = _lane_bcast(val16, t)
                    kk = gg * L + t
                    for j in range(nj):
                        sl = pl.ds(j * L, L)
                        rows[b][kk, sl] = rows[b][kk, sl] * vv
                return carry

            lax.fori_loop(0, k // L, grp, 0)

        start_edges(0, 0)
        start_edges(1, 1)

        def zstore(i, carry):
            for j in range(nj):
                zero_v[i, pl.ds(j * L, L)] = jnp.zeros((L,), jnp.float32)
            return carry

        lax.fori_loop(0, zr, zstore, 0)
        for t in range(rps // zr):
            pltpu.sync_copy(zero_v, acc.at[pl.ds(s * rps + t * zr, zr)])
        plsc.subcore_barrier()
        wait_edges(0)
        shift_idx(0)
        start_gather(0, 0)

        def chunk2(g2, carry):
            for b in (0, 1):
                g = 2 * g2 + b
                ob = 1 - b

                @pl.when(g >= 1)
                def _():
                    wait_scatter(ob)

                @pl.when(g + 1 < nch)
                def _():
                    wait_edges(ob)
                    shift_idx(g + 1)
                    start_gather(g + 1, ob)

                @pl.when(g + 2 < nch)
                def _():
                    start_edges(g + 2, b)

                wait_gather(b)
                scale(g, b)
                start_scatter(g, b)
            return carry

        lax.fori_loop(0, nch // 2, chunk2, 0)
        wait_scatter(1)
        plsc.subcore_barrier()

        pltpu.sync_copy(acc.at[pl.ds(s * rps, rps)],
                        out_ref.at[pl.ds(c * npad + s * rps, rps)])

    out = spmm_kernel(h_stack, rowcol, vals_stack)
    return out, npad


def _fuse_body(lo_ref, hi_ref, x_ref, wm_ref, avl_ref, avh_ref, avm_ref,
               att_ref, out_ref):
    lo = jnp.maximum(lo_ref[...], 0.0)
    hi = jnp.maximum(hi_ref[...], 0.0)
    m = jnp.maximum(
        jnp.dot(x_ref[...], wm_ref[...], preferred_element_type=jnp.float32),
        0.0)

    l0 = jnp.sum(lo * avl_ref[...], axis=1, keepdims=True)
    l1 = jnp.sum(hi * avh_ref[...], axis=1, keepdims=True)
    l2 = jnp.sum(m * avm_ref[...], axis=1, keepdims=True)
    s0 = 1.0 / (1.0 + jnp.exp(-l0))
    s1 = 1.0 / (1.0 + jnp.exp(-l1))
    s2 = 1.0 / (1.0 + jnp.exp(-l2))

    inv_t = 1.0 / 3.0
    z = [(s0 * att_ref[0, jj] + s1 * att_ref[1, jj] + s2 * att_ref[2, jj])
         * inv_t for jj in range(3)]
    zm = jnp.maximum(jnp.maximum(z[0], z[1]), z[2])
    e0 = jnp.exp(z[0] - zm)
    e1 = jnp.exp(z[1] - zm)
    e2 = jnp.exp(z[2] - zm)
    scale = 3.0 / (e0 + e1 + e2)
    out_ref[...] = scale * (e0 * lo + e1 * hi + e2 * m)


def _fuse(lohi, npad, x, w_mlp, av_low, av_high, av_mlp, att_vec):
    n, d = x.shape
    b = _largest_div(math.gcd(n, npad), 1024)
    grid = (n // b,)
    blk = pl.BlockSpec((b, d), lambda i: (i, 0))
    hiblk = pl.BlockSpec((b, d), lambda i, _nb=npad // b: (_nb + i, 0))
    wblk = pl.BlockSpec((d, d), lambda i: (0, 0))
    vblk = pl.BlockSpec((1, d), lambda i: (0, 0))
    ablk = pl.BlockSpec(memory_space=pltpu.SMEM)
    return pl.pallas_call(
        _fuse_body,
        grid=grid,
        in_specs=[blk, hiblk, blk, wblk, vblk, vblk, vblk, ablk],
        out_specs=blk,
        out_shape=jax.ShapeDtypeStruct((n, d), jnp.float32),
    )(lohi, lohi, x, w_mlp, av_low.reshape(1, d), av_high.reshape(1, d),
      av_mlp.reshape(1, d), att_vec)


def kernel(input, edge_index, adj_low_vals, adj_high_vals, weight_low,
           weight_high, weight_mlp, att_vec_low, att_vec_high, att_vec_mlp,
           att_vec):
    h_stack = _projections(input, weight_low, weight_high)
    vals_stack = jnp.concatenate([adj_low_vals, adj_high_vals])
    rowcol = edge_index.reshape(-1)
    lohi, npad = _spmm_sc(h_stack, rowcol, vals_stack)
    return _fuse(lohi, npad, input, weight_mlp, att_vec_low, att_vec_high,
                 att_vec_mlp, att_vec)


# speedup vs baseline: 2.0440x; 2.0440x over previous
import functools
import math

import jax
import jax.numpy as jnp
from jax import lax
from jax.experimental import pallas as pl
from jax.experimental.pallas import tpu as pltpu
from jax.experimental.pallas import tpu_sc as plsc

NC = 2
NS = 16
L = 16


def _largest_div(n, cap, mult=8):
    for b in range(min(n, cap), 0, -1):
        if n % b == 0 and b % mult == 0:
            return b
    raise ValueError((n, cap, mult))


def _proj_body(x_ref, w_ref, h_ref):
    h_ref[...] = jnp.dot(x_ref[...], w_ref[0],
                         preferred_element_type=jnp.float32)


def _projections(x, w_low, w_high):
    n, d = x.shape
    b = _largest_div(n, 1024)
    nb = n // b
    w_stack = jnp.stack([w_low, w_high])
    return pl.pallas_call(
        _proj_body,
        grid=(2, nb),
        in_specs=[pl.BlockSpec((b, d), lambda p, i: (i, 0)),
                  pl.BlockSpec((1, d, d), lambda p, i: (p, 0, 0))],
        out_specs=pl.BlockSpec((b, d), lambda p, i: (p * nb + i, 0)),
        out_shape=jax.ShapeDtypeStruct((2 * n, d), jnp.float32),
    )(x, w_stack)


def _lane_bcast(vec, t):
    return lax.gather(
        vec, jnp.full((L, 1), t, jnp.int32),
        lax.GatherDimensionNumbers(offset_dims=(), collapsed_slice_dims=(0,),
                                   start_index_map=(0,)),
        slice_sizes=(1,),
        mode=lax.GatherScatterMode.PROMISE_IN_BOUNDS)


def _spmm_sc(h_stack, rowcol, vals_stack):
    n2, d = h_stack.shape
    n = n2 // 2
    e = rowcol.shape[0] // 2
    ep = e // NS
    k = _largest_div(ep, 128)
    nch = ep // k
    npad = -(-n // (NS * 8)) * (NS * 8)
    rps = npad // NS
    zr = _largest_div(rps, 64)
    nj = d // L

    mesh = plsc.VectorSubcoreMesh(core_axis_name="c", subcore_axis_name="s",
                                  num_cores=NC, num_subcores=NS)

    @functools.partial(
        pl.kernel,
        out_type=jax.ShapeDtypeStruct((2, npad, d), jnp.float32),
        mesh=mesh,
        scratch_types=[
            pltpu.VMEM((4, k), jnp.int32),
            pltpu.VMEM((4, k), jnp.int32),
            pltpu.VMEM((4, k), jnp.float32),
            pltpu.VMEM((k, d), jnp.float32),
            pltpu.VMEM((k, d), jnp.float32),
            pltpu.VMEM((zr, d), jnp.float32),
            pltpu.VMEM_SHARED((npad, d), jnp.float32),
            pltpu.SemaphoreType.DMA,
            pltpu.SemaphoreType.DMA,
            pltpu.SemaphoreType.DMA,
            pltpu.SemaphoreType.DMA,
            pltpu.SemaphoreType.DMA,
            pltpu.SemaphoreType.DMA,
        ],
    )
    def spmm_kernel(h_ref, rowcol_ref, vals_ref, out_ref,
                    idx4, rdx4, val4, rows0, rows1,
                    zero_v, acc, esem0, esem1, gsem0, gsem1, ssem0, ssem1):
        c = lax.axis_index("c")
        s = lax.axis_index("s")
        rows = (rows0, rows1)
        esem = (esem0, esem1)
        gsem = (gsem0, gsem1)
        ssem = (ssem0, ssem1)

        def start_edges(g, b):
            base = s * ep + g * k
            q = lax.rem(g, 4)
            pltpu.async_copy(rowcol_ref.at[pl.ds(e + base, k)], idx4.at[q],
                             esem[b])
            pltpu.async_copy(rowcol_ref.at[pl.ds(base, k)], rdx4.at[q],
                             esem[b])
            pltpu.async_copy(vals_ref.at[pl.ds(c * e + base, k)], val4.at[q],
                             esem[b])

        def wait_edges(b):
            for dst in (idx4, rdx4, val4):
                pltpu.make_async_copy(rowcol_ref.at[pl.ds(0, k)], dst.at[0],
                                      esem[b]).wait()

        def shift_idx(g):
            q = lax.rem(g, 4)
            off = c * n

            def sbody(gg, carry):
                sl = pl.ds(gg * L, L)
                idx4[q, sl] = idx4[q, sl] + off
                return carry

            lax.fori_loop(0, k // L, sbody, 0)

        def start_gather(g, b):
            pltpu.async_copy(h_ref.at[idx4.at[lax.rem(g, 4)]], rows[b],
                             gsem[b])

        def wait_gather(b):
            pltpu.make_async_copy(h_ref.at[idx4.at[0]], rows[b],
                                  gsem[b]).wait()

        def start_scatter(g, b):
            pltpu.async_copy(rows[b], acc.at[rdx4.at[lax.rem(g, 4)]],
                             ssem[b], add=True)

        def wait_scatter(b):
            pltpu.make_async_copy(rows[b], acc.at[rdx4.at[0]], ssem[b]).wait()

        def scale(g, b):
            q = lax.rem(g, 4)

            def grp(gg, carry):
                val16 = val4[q, pl.ds(gg * L, L)]
                for t in range(L):
                    vv = _lane_bcast(val16, t)
                    kk = gg * L + t
                    for j in range(nj):
                        sl = pl.ds(j * L, L)
                        rows[b][kk, sl] = rows[b][kk, sl] * vv
                return carry

            lax.fori_loop(0, k // L, grp, 0)

        start_edges(0, 0)
        start_edges(1, 1)

        def zstore(i, carry):
            for j in range(nj):
                zero_v[i, pl.ds(j * L, L)] = jnp.zeros((L,), jnp.float32)
            return carry

        lax.fori_loop(0, zr, zstore, 0)
        for t in range(rps // zr):
            pltpu.sync_copy(zero_v, acc.at[pl.ds(s * rps + t * zr, zr)])
        plsc.subcore_barrier()
        wait_edges(0)
        shift_idx(0)
        start_gather(0, 0)

        def chunk2(g2, carry):
            for b in (0, 1):
                g = 2 * g2 + b
                ob = 1 - b

                @pl.when(g >= 1)
                def _():
                    wait_scatter(ob)

                @pl.when(g + 1 < nch)
                def _():
                    wait_edges(ob)
                    shift_idx(g + 1)
                    start_gather(g + 1, ob)

                @pl.when(g + 2 < nch)
                def _():
                    start_edges(g + 2, b)

                wait_gather(b)
                scale(g, b)
                start_scatter(g, b)
            return carry

        lax.fori_loop(0, nch // 2, chunk2, 0)
        wait_scatter(1)
        plsc.subcore_barrier()

        pltpu.sync_copy(acc.at[pl.ds(s * rps, rps)],
                        out_ref.at[c, pl.ds(s * rps, rps)])

    out = spmm_kernel(h_stack, rowcol, vals_stack)
    return out


def _fuse_body(lo_ref, hi_ref, x_ref, wm_ref, avl_ref, avh_ref, avm_ref,
               att_ref, out_ref):
    lo = jnp.maximum(lo_ref[0], 0.0)
    hi = jnp.maximum(hi_ref[0], 0.0)
    m = jnp.maximum(
        jnp.dot(x_ref[...], wm_ref[...], preferred_element_type=jnp.float32),
        0.0)

    l0 = jnp.sum(lo * avl_ref[...], axis=1, keepdims=True)
    l1 = jnp.sum(hi * avh_ref[...], axis=1, keepdims=True)
    l2 = jnp.sum(m * avm_ref[...], axis=1, keepdims=True)
    s0 = 1.0 / (1.0 + jnp.exp(-l0))
    s1 = 1.0 / (1.0 + jnp.exp(-l1))
    s2 = 1.0 / (1.0 + jnp.exp(-l2))

    inv_t = 1.0 / 3.0
    z = [(s0 * att_ref[0, jj] + s1 * att_ref[1, jj] + s2 * att_ref[2, jj])
         * inv_t for jj in range(3)]
    zm = jnp.maximum(jnp.maximum(z[0], z[1]), z[2])
    e0 = jnp.exp(z[0] - zm)
    e1 = jnp.exp(z[1] - zm)
    e2 = jnp.exp(z[2] - zm)
    scale = 3.0 / (e0 + e1 + e2)
    out_ref[...] = scale * (e0 * lo + e1 * hi + e2 * m)


def _fuse(lohi, x, w_mlp, av_low, av_high, av_mlp, att_vec):
    n, d = x.shape
    b = _largest_div(n, 1024)
    grid = (n // b,)
    blk = pl.BlockSpec((b, d), lambda i: (i, 0))
    loblk = pl.BlockSpec((1, b, d), lambda i: (0, i, 0))
    hiblk = pl.BlockSpec((1, b, d), lambda i: (1, i, 0))
    wblk = pl.BlockSpec((d, d), lambda i: (0, 0))
    vblk = pl.BlockSpec((1, d), lambda i: (0, 0))
    ablk = pl.BlockSpec(memory_space=pltpu.SMEM)
    return pl.pallas_call(
        _fuse_body,
        grid=grid,
        in_specs=[loblk, hiblk, blk, wblk, vblk, vblk, vblk, ablk],
        out_specs=blk,
        out_shape=jax.ShapeDtypeStruct((n, d), jnp.float32),
    )(lohi, lohi, x, w_mlp, av_low.reshape(1, d), av_high.reshape(1, d),
      av_mlp.reshape(1, d), att_vec)


def kernel(input, edge_index, adj_low_vals, adj_high_vals, weight_low,
           weight_high, weight_mlp, att_vec_low, att_vec_high, att_vec_mlp,
           att_vec):
    h_stack = _projections(input, weight_low, weight_high)
    vals_stack = jnp.concatenate([adj_low_vals, adj_high_vals])
    rowcol = edge_index.reshape(-1)
    lohi = _spmm_sc(h_stack, rowcol, vals_stack)
    return _fuse(lohi, input, weight_mlp, att_vec_low, att_vec_high,
                 att_vec_mlp, att_vec)


# speedup vs baseline: 2.3034x; 1.1269x over previous
import functools
import math

import jax
import jax.numpy as jnp
from jax import lax
from jax.experimental import pallas as pl
from jax.experimental.pallas import tpu as pltpu
from jax.experimental.pallas import tpu_sc as plsc

NC = 2
NS = 16
L = 16


def _largest_div(n, cap, mult=8):
    for b in range(min(n, cap), 0, -1):
        if n % b == 0 and b % mult == 0:
            return b
    raise ValueError((n, cap, mult))


def _proj_body(x_ref, w_ref, h_ref):
    h_ref[...] = jnp.dot(x_ref[...], w_ref[0],
                         preferred_element_type=jnp.float32)


def _projections(x, w_low, w_high):
    n, d = x.shape
    b = _largest_div(n, 1024)
    nb = n // b
    w_stack = jnp.stack([w_low, w_high])
    return pl.pallas_call(
        _proj_body,
        grid=(2, nb),
        in_specs=[pl.BlockSpec((b, d), lambda p, i: (i, 0)),
                  pl.BlockSpec((1, d, d), lambda p, i: (p, 0, 0))],
        out_specs=pl.BlockSpec((b, d), lambda p, i: (p * nb + i, 0)),
        out_shape=jax.ShapeDtypeStruct((2 * n, d), jnp.float32),
    )(x, w_stack)


def _lane_bcast(vec, t):
    return lax.gather(
        vec, jnp.full((L, 1), t, jnp.int32),
        lax.GatherDimensionNumbers(offset_dims=(), collapsed_slice_dims=(0,),
                                   start_index_map=(0,)),
        slice_sizes=(1,),
        mode=lax.GatherScatterMode.PROMISE_IN_BOUNDS)


def _spmm_sc(h_stack, rowcol, vals_stack):
    n2, d = h_stack.shape
    n = n2 // 2
    e = rowcol.shape[0] // 2
    ep = e // NS
    k = _largest_div(ep, 128)
    nch = ep // k
    npad = -(-n // (NS * 8)) * (NS * 8)
    rps = npad // NS
    zr = _largest_div(rps, 16)
    nj = d // L

    mesh = plsc.VectorSubcoreMesh(core_axis_name="c", subcore_axis_name="s",
                                  num_cores=NC, num_subcores=NS)

    @functools.partial(
        pl.kernel,
        out_type=jax.ShapeDtypeStruct((2, npad, d), jnp.float32),
        mesh=mesh,
        scratch_types=[
            pltpu.VMEM((8, k), jnp.int32),
            pltpu.VMEM((8, k), jnp.int32),
            pltpu.VMEM((8, k), jnp.float32),
            pltpu.VMEM((k, d), jnp.float32),
            pltpu.VMEM((k, d), jnp.float32),
            pltpu.VMEM((k, d), jnp.float32),
            pltpu.VMEM((k, d), jnp.float32),
            pltpu.VMEM((zr, d), jnp.float32),
            pltpu.VMEM_SHARED((npad, d), jnp.float32),
            [pltpu.SemaphoreType.DMA] * 4,
            [pltpu.SemaphoreType.DMA] * 4,
            [pltpu.SemaphoreType.DMA] * 4,
        ],
    )
    def spmm_kernel(h_ref, rowcol_ref, vals_ref, out_ref,
                    idx8, rdx8, val8, rows0, rows1, rows2, rows3,
                    zero_v, acc, esem, gsem, ssem):
        c = lax.axis_index("c")
        s = lax.axis_index("s")
        rows = (rows0, rows1, rows2, rows3)

        def start_edges(g, si):
            base = s * ep + g * k
            q = lax.rem(g, 8)
            sem = esem[si]
            pltpu.async_copy(rowcol_ref.at[pl.ds(e + base, k)], idx8.at[q],
                             sem)
            pltpu.async_copy(rowcol_ref.at[pl.ds(base, k)], rdx8.at[q], sem)
            pltpu.async_copy(vals_ref.at[pl.ds(c * e + base, k)], val8.at[q],
                             sem)

        def wait_edges(b):
            for dst in (idx8, rdx8, val8):
                pltpu.make_async_copy(rowcol_ref.at[pl.ds(0, k)], dst.at[0],
                                      esem[b]).wait()

        def shift_idx(g):
            q = lax.rem(g, 8)
            off = c * n

            def sbody(gg, carry):
                sl = pl.ds(gg * L, L)
                idx8[q, sl] = idx8[q, sl] + off
                return carry

            lax.fori_loop(0, k // L, sbody, 0)

        def start_gather(g, b):
            pltpu.async_copy(h_ref.at[idx8.at[lax.rem(g, 8)]], rows[b],
                             gsem[b])

        def wait_gather(b):
            pltpu.make_async_copy(h_ref.at[idx8.at[0]], rows[b],
                                  gsem[b]).wait()

        def start_scatter(g, b):
            pltpu.async_copy(rows[b], acc.at[rdx8.at[lax.rem(g, 8)]],
                             ssem[b], add=True)

        def wait_scatter(b):
            pltpu.make_async_copy(rows[b], acc.at[rdx8.at[0]], ssem[b]).wait()

        def scale(g, b):
            q = lax.rem(g, 8)

            def grp(gg, carry):
                val16 = val8[q, pl.ds(gg * L, L)]
                for t in range(L):
                    vv = _lane_bcast(val16, t)
                    kk = gg * L + t
                    for j in range(nj):
                        sl = pl.ds(j * L, L)
                        rows[b][kk, sl] = rows[b][kk, sl] * vv
                return carry

            lax.fori_loop(0, k // L, grp, 0)

        def maybe(cond, fn):
            if isinstance(cond, bool):
                if cond:
                    fn()
            else:
                pl.when(cond)(lambda: (fn(), None)[1])

        def step(g, b):
            nb = (b + 1) % 4
            maybe(g >= 3, lambda: wait_scatter(nb))
            maybe(g + 1 < nch, lambda: (wait_edges(nb), shift_idx(g + 1),
                                        start_gather(g + 1, nb)))
            maybe(g + 3 < nch, lambda: start_edges(g + 3, (b + 3) % 4))
            wait_gather(b)
            scale(g, b)
            start_scatter(g, b)

        for g0 in range(3):
            start_edges(g0, g0 % 4)

        def zstore(i, carry):
            for j in range(nj):
                zero_v[i, pl.ds(j * L, L)] = jnp.zeros((L,), jnp.float32)
            return carry

        lax.fori_loop(0, zr, zstore, 0)
        for t in range(rps // zr):
            pltpu.sync_copy(zero_v, acc.at[pl.ds(s * rps + t * zr, zr)])
        plsc.subcore_barrier()
        wait_edges(0)
        shift_idx(0)
        start_gather(0, 0)

        def chunk4(g4, carry):
            for b in (0, 1, 2, 3):
                step(4 * g4 + b, b)
            return carry

        nmain = (nch // 4) * 4
        lax.fori_loop(0, nmain // 4, chunk4, 0)
        for g in range(nmain, nch):
            step(g, g % 4)
        for g in range(nch - 3, nch):
            wait_scatter(g % 4)
        plsc.subcore_barrier()

        pltpu.sync_copy(acc.at[pl.ds(s * rps, rps)],
                        out_ref.at[c, pl.ds(s * rps, rps)])

    out = spmm_kernel(h_stack, rowcol, vals_stack)
    return out


def _fuse_body(lo_ref, hi_ref, x_ref, wm_ref, avl_ref, avh_ref, avm_ref,
               att_ref, out_ref):
    lo = jnp.maximum(lo_ref[0], 0.0)
    hi = jnp.maximum(hi_ref[0], 0.0)
    m = jnp.maximum(
        jnp.dot(x_ref[...], wm_ref[...], preferred_element_type=jnp.float32),
        0.0)

    l0 = jnp.sum(lo * avl_ref[...], axis=1, keepdims=True)
    l1 = jnp.sum(hi * avh_ref[...], axis=1, keepdims=True)
    l2 = jnp.sum(m * avm_ref[...], axis=1, keepdims=True)
    s0 = 1.0 / (1.0 + jnp.exp(-l0))
    s1 = 1.0 / (1.0 + jnp.exp(-l1))
    s2 = 1.0 / (1.0 + jnp.exp(-l2))

    inv_t = 1.0 / 3.0
    z = [(s0 * att_ref[0, jj] + s1 * att_ref[1, jj] + s2 * att_ref[2, jj])
         * inv_t for jj in range(3)]
    zm = jnp.maximum(jnp.maximum(z[0], z[1]), z[2])
    e0 = jnp.exp(z[0] - zm)
    e1 = jnp.exp(z[1] - zm)
    e2 = jnp.exp(z[2] - zm)
    scale = 3.0 / (e0 + e1 + e2)
    out_ref[...] = scale * (e0 * lo + e1 * hi + e2 * m)


def _fuse(lohi, x, w_mlp, av_low, av_high, av_mlp, att_vec):
    n, d = x.shape
    b = _largest_div(n, 1024)
    grid = (n // b,)
    blk = pl.BlockSpec((b, d), lambda i: (i, 0))
    loblk = pl.BlockSpec((1, b, d), lambda i: (0, i, 0))
    hiblk = pl.BlockSpec((1, b, d), lambda i: (1, i, 0))
    wblk = pl.BlockSpec((d, d), lambda i: (0, 0))
    vblk = pl.BlockSpec((1, d), lambda i: (0, 0))
    ablk = pl.BlockSpec(memory_space=pltpu.SMEM)
    return pl.pallas_call(
        _fuse_body,
        grid=grid,
        in_specs=[loblk, hiblk, blk, wblk, vblk, vblk, vblk, ablk],
        out_specs=blk,
        out_shape=jax.ShapeDtypeStruct((n, d), jnp.float32),
    )(lohi, lohi, x, w_mlp, av_low.reshape(1, d), av_high.reshape(1, d),
      av_mlp.reshape(1, d), att_vec)


def kernel(input, edge_index, adj_low_vals, adj_high_vals, weight_low,
           weight_high, weight_mlp, att_vec_low, att_vec_high, att_vec_mlp,
           att_vec):
    h_stack = _projections(input, weight_low, weight_high)
    vals_stack = jnp.concatenate([adj_low_vals, adj_high_vals])
    rowcol = edge_index.reshape(-1)
    lohi = _spmm_sc(h_stack, rowcol, vals_stack)
    return _fuse(lohi, input, weight_mlp, att_vec_low, att_vec_high,
                 att_vec_mlp, att_vec)


# speedup vs baseline: 2.3370x; 1.0146x over previous
import functools
import math

import jax
import jax.numpy as jnp
from jax import lax
from jax.experimental import pallas as pl
from jax.experimental.pallas import tpu as pltpu
from jax.experimental.pallas import tpu_sc as plsc

NC = 2
NS = 16
L = 16


def _largest_div(n, cap, mult=8):
    for b in range(min(n, cap), 0, -1):
        if n % b == 0 and b % mult == 0:
            return b
    raise ValueError((n, cap, mult))


def _proj_body(x_ref, w_ref, h_ref):
    h_ref[...] = jnp.dot(x_ref[...], w_ref[0],
                         preferred_element_type=jnp.float32)


def _projections(x, w_low, w_high):
    n, d = x.shape
    b = _largest_div(n, 1024)
    nb = n // b
    w_stack = jnp.stack([w_low, w_high])
    return pl.pallas_call(
        _proj_body,
        grid=(2, nb),
        in_specs=[pl.BlockSpec((b, d), lambda p, i: (i, 0)),
                  pl.BlockSpec((1, d, d), lambda p, i: (p, 0, 0))],
        out_specs=pl.BlockSpec((b, d), lambda p, i: (p * nb + i, 0)),
        out_shape=jax.ShapeDtypeStruct((2 * n, d), jnp.float32),
    )(x, w_stack)


def _lane_bcast(vec, t):
    return lax.gather(
        vec, jnp.full((L, 1), t, jnp.int32),
        lax.GatherDimensionNumbers(offset_dims=(), collapsed_slice_dims=(0,),
                                   start_index_map=(0,)),
        slice_sizes=(1,),
        mode=lax.GatherScatterMode.PROMISE_IN_BOUNDS)


def _spmm_sc(h_stack, rowcol, vals_stack):
    n2, d = h_stack.shape
    n = n2 // 2
    e = rowcol.shape[0] // 2
    ep = e // NS
    k = _largest_div(ep, 128)
    nch = ep // k
    npad = -(-n // (NS * 8)) * (NS * 8)
    rps = npad // NS
    zr = _largest_div(rps, 16)
    nj = d // L

    mesh = plsc.VectorSubcoreMesh(core_axis_name="c", subcore_axis_name="s",
                                  num_cores=NC, num_subcores=NS)

    @functools.partial(
        pl.kernel,
        out_type=jax.ShapeDtypeStruct((2, npad, d), jnp.float32),
        mesh=mesh,
        scratch_types=[
            pltpu.VMEM((8, k), jnp.int32),
            pltpu.VMEM((8, k), jnp.int32),
            pltpu.VMEM((8, k), jnp.float32),
            pltpu.VMEM((k, d), jnp.float32),
            pltpu.VMEM((k, d), jnp.float32),
            pltpu.VMEM((k, d), jnp.float32),
            pltpu.VMEM((k, d), jnp.float32),
            pltpu.VMEM((zr, d), jnp.float32),
            pltpu.VMEM_SHARED((npad, d), jnp.float32),
            [pltpu.SemaphoreType.DMA] * 4,
            [pltpu.SemaphoreType.DMA] * 4,
            [pltpu.SemaphoreType.DMA] * 4,
        ],
    )
    def spmm_kernel(h_ref, rowcol_ref, vals_ref, out_ref,
                    idx8, rdx8, val8, rows0, rows1, rows2, rows3,
                    zero_v, acc, esem, gsem, ssem):
        c = lax.axis_index("c")
        s = lax.axis_index("s")
        rows = (rows0, rows1, rows2, rows3)

        def start_edges(g, si):
            base = s * ep + g * k
            q = lax.rem(g, 8)
            sem = esem[si]
            pltpu.async_copy(rowcol_ref.at[pl.ds(e + base, k)], idx8.at[q],
                             sem)
            pltpu.async_copy(rowcol_ref.at[pl.ds(base, k)], rdx8.at[q], sem)
            pltpu.async_copy(vals_ref.at[pl.ds(c * e + base, k)], val8.at[q],
                             sem)

        def wait_edges(b):
            for dst in (idx8, rdx8, val8):
                pltpu.make_async_copy(rowcol_ref.at[pl.ds(0, k)], dst.at[0],
                                      esem[b]).wait()

        def shift_idx(g):
            q = lax.rem(g, 8)
            off = c * n

            def sbody(gg, carry):
                sl = pl.ds(gg * L, L)
                idx8[q, sl] = idx8[q, sl] + off
                return carry

            lax.fori_loop(0, k // L, sbody, 0)

        def start_gather(g, b):
            pltpu.async_copy(h_ref.at[idx8.at[lax.rem(g, 8)]], rows[b],
                             gsem[b])

        def wait_gather(b):
            pltpu.make_async_copy(h_ref.at[idx8.at[0]], rows[b],
                                  gsem[b]).wait()

        def start_scatter(g, b):
            pltpu.async_copy(rows[b], acc.at[rdx8.at[lax.rem(g, 8)]],
                             ssem[b], add=True)

        def wait_scatter(b):
            pltpu.make_async_copy(rows[b], acc.at[rdx8.at[0]], ssem[b]).wait()

        def scale(g, b):
            q = lax.rem(g, 8)

            def grp(gg, carry):
                val16 = val8[q, pl.ds(gg * L, L)]
                for t in range(L):
                    vv = _lane_bcast(val16, t)
                    kk = gg * L + t
                    for j in range(nj):
                        sl = pl.ds(j * L, L)
                        rows[b][kk, sl] = rows[b][kk, sl] * vv
                return carry

            lax.fori_loop(0, k // L, grp, 0)

        def maybe(cond, fn):
            if isinstance(cond, bool):
                if cond:
                    fn()
            else:
                pl.when(cond)(lambda: (fn(), None)[1])

        def step(g, b):
            nb = (b + 1) % 4
            maybe(g >= 3, lambda: wait_scatter(nb))
            maybe(g + 1 < nch, lambda: (wait_edges(nb), shift_idx(g + 1),
                                        start_gather(g + 1, nb)))
            maybe(g + 3 < nch, lambda: start_edges(g + 3, (b + 3) % 4))
            wait_gather(b)
            scale(g, b)
            start_scatter(g, b)

        for g0 in range(3):
            start_edges(g0, g0 % 4)

        def zstore(i, carry):
            for j in range(nj):
                zero_v[i, pl.ds(j * L, L)] = jnp.zeros((L,), jnp.float32)
            return carry

        lax.fori_loop(0, zr, zstore, 0)
        for t in range(rps // zr):
            pltpu.async_copy(zero_v, acc.at[pl.ds(s * rps + t * zr, zr)],
                             gsem[3])
        for t in range(rps // zr):
            pltpu.make_async_copy(zero_v, acc.at[pl.ds(s * rps, zr)],
                                  gsem[3]).wait()
        plsc.subcore_barrier()
        wait_edges(0)
        shift_idx(0)
        start_gather(0, 0)

        def chunk4(g4, carry):
            for b in (0, 1, 2, 3):
                step(4 * g4 + b, b)
            return carry

        nmain = (nch // 4) * 4
        lax.fori_loop(0, nmain // 4, chunk4, 0)
        for g in range(nmain, nch):
            step(g, g % 4)
        for g in range(nch - 3, nch):
            wait_scatter(g % 4)
        plsc.subcore_barrier()

        pltpu.sync_copy(acc.at[pl.ds(s * rps, rps)],
                        out_ref.at[c, pl.ds(s * rps, rps)])

    out = spmm_kernel(h_stack, rowcol, vals_stack)
    return out


def _fuse_body(lo_ref, hi_ref, x_ref, wm_ref, avl_ref, avh_ref, avm_ref,
               att_ref, out_ref):
    lo = jnp.maximum(lo_ref[0], 0.0)
    hi = jnp.maximum(hi_ref[0], 0.0)
    m = jnp.maximum(
        jnp.dot(x_ref[...], wm_ref[...], preferred_element_type=jnp.float32),
        0.0)

    l0 = jnp.sum(lo * avl_ref[...], axis=1, keepdims=True)
    l1 = jnp.sum(hi * avh_ref[...], axis=1, keepdims=True)
    l2 = jnp.sum(m * avm_ref[...], axis=1, keepdims=True)
    s0 = 1.0 / (1.0 + jnp.exp(-l0))
    s1 = 1.0 / (1.0 + jnp.exp(-l1))
    s2 = 1.0 / (1.0 + jnp.exp(-l2))

    inv_t = 1.0 / 3.0
    z = [(s0 * att_ref[0, jj] + s1 * att_ref[1, jj] + s2 * att_ref[2, jj])
         * inv_t for jj in range(3)]
    zm = jnp.maximum(jnp.maximum(z[0], z[1]), z[2])
    e0 = jnp.exp(z[0] - zm)
    e1 = jnp.exp(z[1] - zm)
    e2 = jnp.exp(z[2] - zm)
    scale = 3.0 / (e0 + e1 + e2)
    out_ref[...] = scale * (e0 * lo + e1 * hi + e2 * m)


def _fuse(lohi, x, w_mlp, av_low, av_high, av_mlp, att_vec):
    n, d = x.shape
    b = _largest_div(n, 1024)
    grid = (n // b,)
    blk = pl.BlockSpec((b, d), lambda i: (i, 0))
    loblk = pl.BlockSpec((1, b, d), lambda i: (0, i, 0))
    hiblk = pl.BlockSpec((1, b, d), lambda i: (1, i, 0))
    wblk = pl.BlockSpec((d, d), lambda i: (0, 0))
    vblk = pl.BlockSpec((1, d), lambda i: (0, 0))
    ablk = pl.BlockSpec(memory_space=pltpu.SMEM)
    return pl.pallas_call(
        _fuse_body,
        grid=grid,
        in_specs=[loblk, hiblk, blk, wblk, vblk, vblk, vblk, ablk],
        out_specs=blk,
        out_shape=jax.ShapeDtypeStruct((n, d), jnp.float32),
    )(lohi, lohi, x, w_mlp, av_low.reshape(1, d), av_high.reshape(1, d),
      av_mlp.reshape(1, d), att_vec)


def kernel(input, edge_index, adj_low_vals, adj_high_vals, weight_low,
           weight_high, weight_mlp, att_vec_low, att_vec_high, att_vec_mlp,
           att_vec):
    h_stack = _projections(input, weight_low, weight_high)
    vals_stack = jnp.concatenate([adj_low_vals, adj_high_vals])
    rowcol = edge_index.reshape(-1)
    lohi = _spmm_sc(h_stack, rowcol, vals_stack)
    return _fuse(lohi, input, weight_mlp, att_vec_low, att_vec_high,
                 att_vec_mlp, att_vec)


# speedup vs baseline: 2.3387x; 1.0007x over previous
import functools

import jax
import jax.numpy as jnp
from jax import lax
from jax.experimental import pallas as pl
from jax.experimental.pallas import tpu as pltpu
from jax.experimental.pallas import tpu_sc as plsc

NC = 2
NS = 16
L = 16


def _largest_div(n, cap, mult=8):
    for b in range(min(n, cap), 0, -1):
        if n % b == 0 and b % mult == 0:
            return b
    raise ValueError((n, cap, mult))


def _proj_body(x_ref, w_ref, h_ref):
    h_ref[...] = jnp.dot(x_ref[...], w_ref[0],
                         preferred_element_type=jnp.float32)


def _projections(x, w_low, w_high):
    n, d = x.shape
    b = _largest_div(n, 1024)
    nb = n // b
    w_stack = jnp.stack([w_low, w_high])
    return pl.pallas_call(
        _proj_body,
        grid=(2, nb),
        in_specs=[pl.BlockSpec((b, d), lambda p, i: (i, 0)),
                  pl.BlockSpec((1, d, d), lambda p, i: (p, 0, 0))],
        out_specs=pl.BlockSpec((b, d), lambda p, i: (p * nb + i, 0)),
        out_shape=jax.ShapeDtypeStruct((2 * n, d), jnp.float32),
    )(x, w_stack)


def _lane_bcast(vec, t):
    return lax.gather(
        vec, jnp.full((L, 1), t, jnp.int32),
        lax.GatherDimensionNumbers(offset_dims=(), collapsed_slice_dims=(0,),
                                   start_index_map=(0,)),
        slice_sizes=(1,),
        mode=lax.GatherScatterMode.PROMISE_IN_BOUNDS)


def _spmm_sc(h_stack, rowcol, vals_stack):
    n2, d = h_stack.shape
    n = n2 // 2
    e = rowcol.shape[0] // 2
    ep = e // NS
    k = _largest_div(ep, 128)
    nch = ep // k
    npad = -(-n // (NS * 8)) * (NS * 8)
    rps = npad // NS
    zr = _largest_div(rps, 16)
    nj = d // L

    mesh = plsc.VectorSubcoreMesh(core_axis_name="c", subcore_axis_name="s",
                                  num_cores=NC, num_subcores=NS)

    @functools.partial(
        pl.kernel,
        out_type=jax.ShapeDtypeStruct((2, npad, d), jnp.float32),
        mesh=mesh,
        scratch_types=[
            pltpu.VMEM((8, k), jnp.int32),
            pltpu.VMEM((8, k), jnp.int32),
            pltpu.VMEM((8, k), jnp.float32),
            pltpu.VMEM((k, d), jnp.float32),
            pltpu.VMEM((k, d), jnp.float32),
            pltpu.VMEM((k, d), jnp.float32),
            pltpu.VMEM((k, d), jnp.float32),
            pltpu.VMEM((zr, d), jnp.float32),
            pltpu.VMEM_SHARED((npad, d), jnp.float32),
            [pltpu.SemaphoreType.DMA] * 4,
            [pltpu.SemaphoreType.DMA] * 4,
            [pltpu.SemaphoreType.DMA] * 4,
        ],
    )
    def spmm_kernel(h_ref, rowcol_ref, vals_ref, out_ref,
                    idx8, rdx8, val8, rows0, rows1, rows2, rows3,
                    zero_v, acc, esem, gsem, ssem):
        c = lax.axis_index("c")
        s = lax.axis_index("s")
        rows = (rows0, rows1, rows2, rows3)

        def start_edges(g, si):
            base = s * ep + g * k
            q = lax.rem(g, 8)
            sem = esem[si]
            pltpu.async_copy(rowcol_ref.at[pl.ds(e + base, k)], idx8.at[q],
                             sem)
            pltpu.async_copy(rowcol_ref.at[pl.ds(base, k)], rdx8.at[q], sem)
            pltpu.async_copy(vals_ref.at[pl.ds(c * e + base, k)], val8.at[q],
                             sem)

        def wait_edges(b):
            for dst in (idx8, rdx8, val8):
                pltpu.make_async_copy(rowcol_ref.at[pl.ds(0, k)], dst.at[0],
                                      esem[b]).wait()

        def shift_idx(g):
            q = lax.rem(g, 8)
            off = c * n

            def sbody(gg, carry):
                sl = pl.ds(gg * L, L)
                idx8[q, sl] = idx8[q, sl] + off
                return carry

            lax.fori_loop(0, k // L, sbody, 0)

        def start_gather(g, b):
            pltpu.async_copy(h_ref.at[idx8.at[lax.rem(g, 8)]], rows[b],
                             gsem[b])

        def wait_gather(b):
            pltpu.make_async_copy(h_ref.at[idx8.at[0]], rows[b],
                                  gsem[b]).wait()

        def start_scatter(g, b):
            pltpu.async_copy(rows[b], acc.at[rdx8.at[lax.rem(g, 8)]],
                             ssem[b], add=True)

        def wait_scatter(b):
            pltpu.make_async_copy(rows[b], acc.at[rdx8.at[0]], ssem[b]).wait()

        def scale(g, b):
            q = lax.rem(g, 8)

            def grp(gg, carry):
                val16 = val8[q, pl.ds(gg * L, L)]
                for t in range(L):
                    vv = _lane_bcast(val16, t)
                    kk = gg * L + t
                    for j in range(nj):
                        sl = pl.ds(j * L, L)
                        rows[b][kk, sl] = rows[b][kk, sl] * vv
                return carry

            lax.fori_loop(0, k // L, grp, 0)

        def maybe(cond, fn):
            if isinstance(cond, bool):
                if cond:
                    fn()
            else:
                pl.when(cond)(lambda: (fn(), None)[1])

        def step(g, b):
            nb = (b + 1) % 4
            maybe(g >= 3, lambda: wait_scatter(nb))
            maybe(g + 1 < nch, lambda: (wait_edges(nb), shift_idx(g + 1),
                                        start_gather(g + 1, nb)))
            maybe(g + 3 < nch, lambda: start_edges(g + 3, (b + 3) % 4))
            wait_gather(b)
            scale(g, b)
            start_scatter(g, b)

        for g0 in range(3):
            start_edges(g0, g0 % 4)

        def zstore(i, carry):
            for j in range(nj):
                zero_v[i, pl.ds(j * L, L)] = jnp.zeros((L,), jnp.float32)
            return carry

        lax.fori_loop(0, zr, zstore, 0)
        for t in range(rps // zr):
            pltpu.async_copy(zero_v, acc.at[pl.ds(s * rps + t * zr, zr)],
                             gsem[3])
        for t in range(rps // zr):
            pltpu.make_async_copy(zero_v, acc.at[pl.ds(s * rps, zr)],
                                  gsem[3]).wait()
        plsc.subcore_barrier()
        wait_edges(0)
        shift_idx(0)
        start_gather(0, 0)

        def chunk4(g4, carry):
            for b in (0, 1, 2, 3):
                step(4 * g4 + b, b)
            return carry

        nmain = (nch // 4) * 4
        lax.fori_loop(0, nmain // 4, chunk4, 0)
        for g in range(nmain, nch):
            step(g, g % 4)
        for g in range(nch - 3, nch):
            wait_scatter(g % 4)
        plsc.subcore_barrier()

        pltpu.sync_copy(acc.at[pl.ds(s * rps, rps)],
                        out_ref.at[c, pl.ds(s * rps, rps)])

    out = spmm_kernel(h_stack, rowcol, vals_stack)
    return out


def _fuse_body(lo_ref, hi_ref, x_ref, wm_ref, avl_ref, avh_ref, avm_ref,
               att_ref, out_ref):
    lo = jnp.maximum(lo_ref[0], 0.0)
    hi = jnp.maximum(hi_ref[0], 0.0)
    m = jnp.maximum(
        jnp.dot(x_ref[...], wm_ref[...], preferred_element_type=jnp.float32),
        0.0)

    l0 = jnp.sum(lo * avl_ref[...], axis=1, keepdims=True)
    l1 = jnp.sum(hi * avh_ref[...], axis=1, keepdims=True)
    l2 = jnp.sum(m * avm_ref[...], axis=1, keepdims=True)
    s0 = 1.0 / (1.0 + jnp.exp(-l0))
    s1 = 1.0 / (1.0 + jnp.exp(-l1))
    s2 = 1.0 / (1.0 + jnp.exp(-l2))

    inv_t = 1.0 / 3.0
    z = [(s0 * att_ref[0, jj] + s1 * att_ref[1, jj] + s2 * att_ref[2, jj])
         * inv_t for jj in range(3)]
    zm = jnp.maximum(jnp.maximum(z[0], z[1]), z[2])
    e0 = jnp.exp(z[0] - zm)
    e1 = jnp.exp(z[1] - zm)
    e2 = jnp.exp(z[2] - zm)
    scale = 3.0 / (e0 + e1 + e2)
    out_ref[...] = scale * (e0 * lo + e1 * hi + e2 * m)


def _fuse(lohi, x, w_mlp, av_low, av_high, av_mlp, att_vec):
    n, d = x.shape
    b = _largest_div(n, 1024)
    grid = (n // b,)
    blk = pl.BlockSpec((b, d), lambda i: (i, 0))
    loblk = pl.BlockSpec((1, b, d), lambda i: (0, i, 0))
    hiblk = pl.BlockSpec((1, b, d), lambda i: (1, i, 0))
    wblk = pl.BlockSpec((d, d), lambda i: (0, 0))
    vblk = pl.BlockSpec((1, d), lambda i: (0, 0))
    ablk = pl.BlockSpec(memory_space=pltpu.SMEM)
    return pl.pallas_call(
        _fuse_body,
        grid=grid,
        in_specs=[loblk, hiblk, blk, wblk, vblk, vblk, vblk, ablk],
        out_specs=blk,
        out_shape=jax.ShapeDtypeStruct((n, d), jnp.float32),
    )(lohi, lohi, x, w_mlp, av_low.reshape(1, d), av_high.reshape(1, d),
      av_mlp.reshape(1, d), att_vec)


def kernel(input, edge_index, adj_low_vals, adj_high_vals, weight_low,
           weight_high, weight_mlp, att_vec_low, att_vec_high, att_vec_mlp,
           att_vec):
    h_stack = _projections(input, weight_low, weight_high)
    vals_stack = jnp.concatenate([adj_low_vals, adj_high_vals])
    rowcol = edge_index.reshape(-1)
    lohi = _spmm_sc(h_stack, rowcol, vals_stack)
    return _fuse(lohi, input, weight_mlp, att_vec_low, att_vec_high,
                 att_vec_mlp, att_vec)
